# baseline (speedup 1.0000x reference)
import functools

import jax
import jax.numpy as jnp
from jax import lax
from jax.experimental import pallas as pl
from jax.experimental.pallas import tpu as pltpu
from jax.experimental.pallas import tpu_sc as plsc

_BR = 1600
_W = 192
_CH = 40


def _mlp_body(starts_ref, xb, yb, zb, w1x, w1y, b1, w2, b2, z1_out, cnt_out,
              cnt, *, blk0, nblk, q, w):
    i = pl.program_id(0)

    @pl.when(i == 0)
    def _init():
        cnt[...] = jnp.zeros_like(cnt)

    h = jnp.maximum(
        jnp.dot(xb[...], w1x[...], preferred_element_type=jnp.float32)
        + yb[...] * w1y[...] + b1[...], 0.0)
    z1_out[...] = jnp.maximum(
        jnp.dot(h, w2[...], preferred_element_type=jnp.float32) + b2[...], 0.0)

    start = pl.multiple_of(starts_ref[blk0 + i], 8)
    local = zb[0] - start
    br = local.shape[-1]
    oh = (jax.lax.broadcasted_iota(jnp.int32, (w, br), 0)
          == jnp.broadcast_to(local, (w, br))).astype(jnp.float32)
    cnt[pl.ds(start, w), :] = cnt[pl.ds(start, w), :] + jnp.sum(
        oh, axis=1, keepdims=True)

    @pl.when(i == nblk - 1)
    def _done():
        cnt_out[...] = cnt[...][:q]


def _heads_body(sums, cnts, wm, bm, wv, bv, eps,
                out_mean, out_logvar, out_mmb):
    c = jnp.sum(cnts[...], axis=1, keepdims=True)
    pos = c > 0.0
    bmat = jnp.where(pos, sums[...] / jnp.where(pos, c, 1.0), 0.0)
    m = jnp.dot(bmat, wm[...], preferred_element_type=jnp.float32) + bm[...]
    v = jnp.dot(bmat, wv[...], preferred_element_type=jnp.float32) + bv[...]
    out_mean[...] = m
    out_logvar[...] = v
    out_mmb[...] = m + jnp.exp(0.5 * v) * eps[...]


def _segsum_sc(n, q, h2):
    hc = h2 // 2
    ns = 16
    nchunk = n // _CH
    ct = nchunk // ns
    qrows = (q // ns) // 8 * 8
    qtail = q - qrows * ns
    mesh = plsc.VectorSubcoreMesh(core_axis_name="c", subcore_axis_name="s")

    @functools.partial(
        pl.kernel,
        mesh=mesh,
        out_type=jax.ShapeDtypeStruct((q, h2), jnp.float32),
        scratch_types=[
            pltpu.VMEM_SHARED((q, hc), jnp.float32),
            pltpu.VMEM((_CH, hc), jnp.float32),
            pltpu.VMEM((_CH, hc), jnp.float32),
            pltpu.VMEM((1, _CH), jnp.int32),
            pltpu.VMEM((1, _CH), jnp.int32),
            pltpu.SemaphoreType.DMA,
            pltpu.SemaphoreType.DMA,
            pltpu.SemaphoreType.DMA,
            pltpu.SemaphoreType.DMA,
        ],
    )
    def segsum(z1_hbm, z3_hbm, init_hbm, sums_hbm, table,
               rows0, rows1, idx0, idx1, semr0, semr1, semi0, semi1):
        c = lax.axis_index("c")
        s = lax.axis_index("s")
        qr0 = s * qrows

        pltpu.sync_copy(init_hbm.at[pl.ds(qr0, qrows), pl.ds(c * hc, hc)],
                        table.at[pl.ds(qr0, qrows), :])

        @pl.when(s == ns - 1)
        def _():
            pltpu.sync_copy(
                init_hbm.at[pl.ds(ns * qrows, qtail), pl.ds(c * hc, hc)],
                table.at[pl.ds(ns * qrows, qtail), :])

        plsc.subcore_barrier()

        k0 = s * ct
        rows = (rows0, rows1)
        idx = (idx0, idx1)
        semr = (semr0, semr1)
        semi = (semi0, semi1)

        def rows_src(k):
            return z1_hbm.at[pl.ds((k0 + k) * _CH, _CH), pl.ds(c * hc, hc)]

        def idx_src(k):
            return z3_hbm.at[k0 + k]

        def issue(k, b):
            pltpu.async_copy(rows_src(k), rows[b], semr[b])
            pltpu.async_copy(idx_src(k), idx[b], semi[b])

        issue(0, 0)
        issue(1, 1)

        def step(k, carry):
            for b in range(2):
                @pl.when(k % 2 == b)
                def _():
                    pltpu.make_async_copy(rows_src(k), rows[b], semr[b]).wait()
                    pltpu.make_async_copy(idx_src(k), idx[b], semi[b]).wait()
                    pltpu.sync_copy(rows[b], table.at[idx[b].at[0]], add=True)

                    @pl.when(k + 2 < ct)
                    def _():
                        issue(k + 2, b)
            return carry

        lax.fori_loop(0, ct, step, 0)
        plsc.subcore_barrier()

        pltpu.sync_copy(table.at[pl.ds(qr0, qrows), :],
                        sums_hbm.at[pl.ds(qr0, qrows), pl.ds(c * hc, hc)])

        @pl.when(s == ns - 1)
        def _():
            pltpu.sync_copy(
                table.at[pl.ds(ns * qrows, qtail), :],
                sums_hbm.at[pl.ds(ns * qrows, qtail), pl.ds(c * hc, hc)])

    return segsum


def kernel(X, y, Z, W1, b1, W2, b2, Wm, bm, Wv, bv):
    n, in_dim = X.shape
    h1 = W1.shape[1]
    h2 = W2.shape[1]
    d = Wm.shape[1]
    q = 10000
    br = _BR
    nblk = n // br
    w = _W
    qw = q + w

    zi = Z.astype(jnp.int32)
    zr = zi.reshape(nblk, 1, br)
    starts = (zi[::br] // 8) * 8
    w1x = W1[:in_dim]
    w1y = W1[in_dim:in_dim + 1]
    eps = jax.random.normal(jax.random.key(42), (q, d), dtype=jnp.float32)

    nca = n * 16 // 25
    ncs = [nca, n - nca]
    if any(m % br or (m // 16) % _CH for m in ncs):
        ncs = [n // 2, n - n // 2]

    sums = jnp.zeros((q, h2), jnp.float32)
    cnts_list = []
    r0 = 0
    blk0 = 0
    for t, nc in enumerate(ncs):
        nblk_c = nc // br
        grid_spec = pltpu.PrefetchScalarGridSpec(
            num_scalar_prefetch=1,
            grid=(nblk_c,),
            in_specs=[
                pl.BlockSpec((br, in_dim),
                             lambda i, s, b=blk0: (b + i, 0)),
                pl.BlockSpec((br, 1), lambda i, s, b=blk0: (b + i, 0)),
                pl.BlockSpec((1, 1, br),
                             lambda i, s, b=blk0: (b + i, 0, 0)),
                pl.BlockSpec((in_dim, h1), lambda i, s: (0, 0)),
                pl.BlockSpec((1, h1), lambda i, s: (0, 0)),
                pl.BlockSpec((1, h1), lambda i, s: (0, 0)),
                pl.BlockSpec((h1, h2), lambda i, s: (0, 0)),
                pl.BlockSpec((1, h2), lambda i, s: (0, 0)),
            ],
            out_specs=[
                pl.BlockSpec((br, h2), lambda i, s: (i, 0)),
                pl.BlockSpec((q, 1), lambda i, s: (0, 0)),
            ],
            scratch_shapes=[pltpu.VMEM((qw, 1), jnp.float32)],
        )
        z1_t, cnt_t = pl.pallas_call(
            functools.partial(_mlp_body, blk0=blk0, nblk=nblk_c, q=q, w=w),
            grid_spec=grid_spec,
            out_shape=[
                jax.ShapeDtypeStruct((nc, h2), jnp.float32),
                jax.ShapeDtypeStruct((q, 1), jnp.float32),
            ],
            compiler_params=pltpu.CompilerParams(
                dimension_semantics=("arbitrary",)),
        )(starts, X, y, zr, w1x, w1y, b1.reshape(1, h1), W2,
          b2.reshape(1, h2))
        z3d_t = lax.dynamic_slice_in_dim(
            zi, r0, nc).reshape(nc // _CH, 1, _CH)
        sums = _segsum_sc(nc, q, h2)(z1_t, z3d_t, sums)
        cnts_list.append(cnt_t)
        r0 += nc
        blk0 += nblk_c

    cnts = jnp.concatenate(cnts_list, axis=1)
    const = lambda i: (0, 0)
    mean, logvar, mmb = pl.pallas_call(
        _heads_body,
        grid=(1,),
        in_specs=[
            pl.BlockSpec((q, h2), const),
            pl.BlockSpec((q, len(ncs)), const),
            pl.BlockSpec((h2, d), const),
            pl.BlockSpec((1, d), const),
            pl.BlockSpec((h2, d), const),
            pl.BlockSpec((1, d), const),
            pl.BlockSpec((q, d), const),
        ],
        out_specs=[
            pl.BlockSpec((q, d), const),
            pl.BlockSpec((q, d), const),
            pl.BlockSpec((q, d), const),
        ],
        out_shape=[jax.ShapeDtypeStruct((q, d), jnp.float32)] * 3,
    )(sums, cnts, Wm, bm.reshape(1, d), Wv, bv.reshape(1, d), eps)
    return (mean, logvar, mmb)


# speedup vs baseline: 1.0329x; 1.0329x over previous
import functools

import jax
import jax.numpy as jnp
from jax import lax
from jax.experimental import pallas as pl
from jax.experimental.pallas import tpu as pltpu
from jax.experimental.pallas import tpu_sc as plsc

_BR = 1600
_W = 192
_CH = 40
_KC = 2


def _mlp_body(starts_ref, xb, yb, zb, w1x, w1y, b1, w2, b2, z1_out, cnt_out,
              cnt, *, blk0, nblk, q, w):
    i = pl.program_id(0)

    @pl.when(i == 0)
    def _init():
        cnt[...] = jnp.zeros_like(cnt)

    h = jnp.maximum(
        jnp.dot(xb[...], w1x[...], preferred_element_type=jnp.float32)
        + yb[...] * w1y[...] + b1[...], 0.0)
    z1_out[...] = jnp.maximum(
        jnp.dot(h, w2[...], preferred_element_type=jnp.float32) + b2[...], 0.0)

    start = pl.multiple_of(starts_ref[blk0 + i], 8)
    local = zb[0] - start
    br = local.shape[-1]
    oh = (jax.lax.broadcasted_iota(jnp.int32, (w, br), 0)
          == jnp.broadcast_to(local, (w, br))).astype(jnp.float32)
    cnt[pl.ds(start, w), :] = cnt[pl.ds(start, w), :] + jnp.sum(
        oh, axis=1, keepdims=True)

    @pl.when(i == nblk - 1)
    def _done():
        cnt_out[...] = cnt[...][:q]


def _heads_body(sums, cnts, wm, bm, wv, bv, eps,
                out_mean, out_logvar, out_mmb):
    c = jnp.sum(cnts[...], axis=1, keepdims=True)
    pos = c > 0.0
    bmat = jnp.where(pos, sums[...] / jnp.where(pos, c, 1.0), 0.0)
    m = jnp.dot(bmat, wm[...], preferred_element_type=jnp.float32) + bm[...]
    v = jnp.dot(bmat, wv[...], preferred_element_type=jnp.float32) + bv[...]
    out_mean[...] = m
    out_logvar[...] = v
    out_mmb[...] = m + jnp.exp(0.5 * v) * eps[...]


def _segsum_sc(n, q, h2):
    hc = h2 // 2
    ns = 16
    nchunk = n // _CH
    ct = nchunk // ns
    qrows = (q // ns) // 8 * 8
    qtail = q - qrows * ns
    mesh = plsc.VectorSubcoreMesh(core_axis_name="c", subcore_axis_name="s")

    @functools.partial(
        pl.kernel,
        mesh=mesh,
        out_type=jax.ShapeDtypeStruct((q, h2), jnp.float32),
        scratch_types=[
            pltpu.VMEM_SHARED((q, hc), jnp.float32),
            pltpu.VMEM((_CH, hc), jnp.float32),
            pltpu.VMEM((_CH, hc), jnp.float32),
            pltpu.VMEM((1, _CH), jnp.int32),
            pltpu.VMEM((1, _CH), jnp.int32),
            pltpu.SemaphoreType.DMA,
            pltpu.SemaphoreType.DMA,
            pltpu.SemaphoreType.DMA,
            pltpu.SemaphoreType.DMA,
        ],
    )
    def segsum(z1_hbm, z3_hbm, init_hbm, sums_hbm, table,
               rows0, rows1, idx0, idx1, semr0, semr1, semi0, semi1):
        c = lax.axis_index("c")
        s = lax.axis_index("s")
        qr0 = s * qrows

        pltpu.sync_copy(init_hbm.at[pl.ds(qr0, qrows), pl.ds(c * hc, hc)],
                        table.at[pl.ds(qr0, qrows), :])

        @pl.when(s == ns - 1)
        def _():
            pltpu.sync_copy(
                init_hbm.at[pl.ds(ns * qrows, qtail), pl.ds(c * hc, hc)],
                table.at[pl.ds(ns * qrows, qtail), :])

        plsc.subcore_barrier()

        k0 = s * ct
        rows = (rows0, rows1)
        idx = (idx0, idx1)
        semr = (semr0, semr1)
        semi = (semi0, semi1)

        def rows_src(k):
            return z1_hbm.at[pl.ds((k0 + k) * _CH, _CH), pl.ds(c * hc, hc)]

        def idx_src(k):
            return z3_hbm.at[k0 + k]

        def issue(k, b):
            pltpu.async_copy(rows_src(k), rows[b], semr[b])
            pltpu.async_copy(idx_src(k), idx[b], semi[b])

        issue(0, 0)
        issue(1, 1)

        def step(k, carry):
            for b in range(2):
                @pl.when(k % 2 == b)
                def _():
                    pltpu.make_async_copy(rows_src(k), rows[b], semr[b]).wait()
                    pltpu.make_async_copy(idx_src(k), idx[b], semi[b]).wait()
                    pltpu.sync_copy(rows[b], table.at[idx[b].at[0]], add=True)

                    @pl.when(k + 2 < ct)
                    def _():
                        issue(k + 2, b)
            return carry

        lax.fori_loop(0, ct, step, 0)
        plsc.subcore_barrier()

        pltpu.sync_copy(table.at[pl.ds(qr0, qrows), :],
                        sums_hbm.at[pl.ds(qr0, qrows), pl.ds(c * hc, hc)])

        @pl.when(s == ns - 1)
        def _():
            pltpu.sync_copy(
                table.at[pl.ds(ns * qrows, qtail), :],
                sums_hbm.at[pl.ds(ns * qrows, qtail), pl.ds(c * hc, hc)])

    return segsum


def kernel(X, y, Z, W1, b1, W2, b2, Wm, bm, Wv, bv):
    n, in_dim = X.shape
    h1 = W1.shape[1]
    h2 = W2.shape[1]
    d = Wm.shape[1]
    q = 10000
    br = _BR
    nblk = n // br
    w = _W
    qw = q + w

    zi = Z.astype(jnp.int32)
    zr = zi.reshape(nblk, 1, br)
    starts = (zi[::br] // 8) * 8
    w1x = W1[:in_dim]
    w1y = W1[in_dim:in_dim + 1]
    eps = jax.random.normal(jax.random.key(42), (q, d), dtype=jnp.float32)

    nc = n // _KC
    nblk_c = nc // br
    segsum = _segsum_sc(nc, q, h2)

    sums = jnp.zeros((q, h2), jnp.float32)
    cnts_list = []
    for t in range(_KC):
        grid_spec = pltpu.PrefetchScalarGridSpec(
            num_scalar_prefetch=1,
            grid=(nblk_c,),
            in_specs=[
                pl.BlockSpec((br, in_dim),
                             lambda i, s, t=t: (t * nblk_c + i, 0)),
                pl.BlockSpec((br, 1), lambda i, s, t=t: (t * nblk_c + i, 0)),
                pl.BlockSpec((1, 1, br),
                             lambda i, s, t=t: (t * nblk_c + i, 0, 0)),
                pl.BlockSpec((in_dim, h1), lambda i, s: (0, 0)),
                pl.BlockSpec((1, h1), lambda i, s: (0, 0)),
                pl.BlockSpec((1, h1), lambda i, s: (0, 0)),
                pl.BlockSpec((h1, h2), lambda i, s: (0, 0)),
                pl.BlockSpec((1, h2), lambda i, s: (0, 0)),
            ],
            out_specs=[
                pl.BlockSpec((br, h2), lambda i, s: (i, 0)),
                pl.BlockSpec((q, 1), lambda i, s: (0, 0)),
            ],
            scratch_shapes=[pltpu.VMEM((qw, 1), jnp.float32)],
        )
        z1_t, cnt_t = pl.pallas_call(
            functools.partial(_mlp_body, blk0=t * nblk_c, nblk=nblk_c,
                              q=q, w=w),
            grid_spec=grid_spec,
            out_shape=[
                jax.ShapeDtypeStruct((nc, h2), jnp.float32),
                jax.ShapeDtypeStruct((q, 1), jnp.float32),
            ],
            compiler_params=pltpu.CompilerParams(
                dimension_semantics=("arbitrary",)),
        )(starts, X, y, zr, w1x, w1y, b1.reshape(1, h1), W2,
          b2.reshape(1, h2))
        z3d_t = lax.dynamic_slice_in_dim(
            zi, t * nc, nc).reshape(nc // _CH, 1, _CH)
        sums = segsum(z1_t, z3d_t, sums)
        cnts_list.append(cnt_t)

    cnts = jnp.concatenate(cnts_list, axis=1)
    const = lambda i: (0, 0)
    mean, logvar, mmb = pl.pallas_call(
        _heads_body,
        grid=(1,),
        in_specs=[
            pl.BlockSpec((q, h2), const),
            pl.BlockSpec((q, _KC), const),
            pl.BlockSpec((h2, d), const),
            pl.BlockSpec((1, d), const),
            pl.BlockSpec((h2, d), const),
            pl.BlockSpec((1, d), const),
            pl.BlockSpec((q, d), const),
        ],
        out_specs=[
            pl.BlockSpec((q, d), const),
            pl.BlockSpec((q, d), const),
            pl.BlockSpec((q, d), const),
        ],
        out_shape=[jax.ShapeDtypeStruct((q, d), jnp.float32)] * 3,
    )(sums, cnts, Wm, bm.reshape(1, d), Wv, bv.reshape(1, d), eps)
    return (mean, logvar, mmb)
